# baseline (device time: 30148 ns/iter reference)
import jax
import jax.numpy as jnp
from jax import lax
from jax.experimental import pallas as pl
from jax.experimental.pallas import tpu as pltpu


def kernel(x, router, W1, W2):
    t_loc, d = x.shape
    e_loc, _, f = W1.shape
    t = 2 * t_loc

    def body(x_ref, r_ref, w1_ref, w2_ref, out_ref,
             xpeer_ref, rpeer_ref, cb_ref, partial_ref,
             send_sems, recv_sems):
        my_x = lax.axis_index("x")
        my_y = lax.axis_index("y")
        my_z = lax.axis_index("z")
        peer = (1 - my_x, my_y, my_z)
        mesh_t = pl.DeviceIdType.MESH

        barrier_sem = pltpu.get_barrier_semaphore()
        pl.semaphore_signal(barrier_sem, inc=1, device_id=peer,
                            device_id_type=mesh_t)
        pl.semaphore_wait(barrier_sem, 1)

        rdma_x = pltpu.make_async_remote_copy(
            src_ref=x_ref, dst_ref=xpeer_ref,
            send_sem=send_sems.at[0], recv_sem=recv_sems.at[0],
            device_id=peer, device_id_type=mesh_t)
        rdma_x.start()
        rdma_r = pltpu.make_async_remote_copy(
            src_ref=r_ref, dst_ref=rpeer_ref,
            send_sem=send_sems.at[1], recv_sem=recv_sems.at[1],
            device_id=peer, device_id_type=mesh_t)
        rdma_r.start()
        rdma_x.wait()
        rdma_r.wait()

        X = jnp.concatenate([x_ref[...], xpeer_ref[...]], axis=0)
        gates = jnp.concatenate(
            [jnp.dot(X, r_ref[...], preferred_element_type=jnp.float32),
             jnp.dot(X, rpeer_ref[...], preferred_element_type=jnp.float32)],
            axis=1)

        eidx = lax.broadcasted_iota(jnp.int32, (t, 4), 1)
        m1 = jnp.max(gates, axis=1, keepdims=True)
        i1 = jnp.min(jnp.where(gates == m1, eidx, 4), axis=1, keepdims=True)
        masked = jnp.where(eidx == i1, -jnp.inf, gates)
        m2 = jnp.max(masked, axis=1, keepdims=True)
        i2 = jnp.min(jnp.where(masked == m2, eidx, 4), axis=1, keepdims=True)
        b = jnp.exp(m2 - m1)
        w_top = 1.0 / (1.0 + b)
        w_sec = b / (1.0 + b)

        Xb = X.astype(jnp.bfloat16)
        acc = jnp.zeros((t, d), jnp.float32)
        for j in range(e_loc):
            wj = (jnp.where(i1 == j, w_top, 0.0)
                  + jnp.where(i2 == j, w_sec, 0.0))
            h = jnp.maximum(
                jnp.dot(Xb, w1_ref[j].astype(jnp.bfloat16),
                        preferred_element_type=jnp.float32),
                0.0).astype(jnp.bfloat16)
            acc = acc + wj * jnp.dot(
                h, w2_ref[j].astype(jnp.bfloat16),
                preferred_element_type=jnp.float32)

        cb_ref[...] = acc[t_loc:, :]
        rdma_c = pltpu.make_async_remote_copy(
            src_ref=cb_ref, dst_ref=partial_ref,
            send_sem=send_sems.at[2], recv_sem=recv_sems.at[2],
            device_id=peer, device_id_type=mesh_t)
        rdma_c.start()
        rdma_c.wait()

        out_ref[...] = acc[:t_loc, :] + partial_ref[...]

    return pl.pallas_call(
        body,
        out_shape=jax.ShapeDtypeStruct((t_loc, d), jnp.float32),
        in_specs=[pl.BlockSpec(memory_space=pltpu.VMEM)] * 4,
        out_specs=pl.BlockSpec(memory_space=pltpu.VMEM),
        scratch_shapes=[
            pltpu.VMEM((t_loc, d), jnp.float32),
            pltpu.VMEM((d, e_loc), jnp.float32),
            pltpu.VMEM((t_loc, d), jnp.float32),
            pltpu.VMEM((t_loc, d), jnp.float32),
            pltpu.SemaphoreType.DMA((3,)),
            pltpu.SemaphoreType.DMA((3,)),
        ],
        compiler_params=pltpu.CompilerParams(collective_id=0),
    )(x, router, W1, W2)


# device time: 27810 ns/iter; 1.0841x vs baseline; 1.0841x over previous
import os

import jax
import jax.numpy as jnp
from jax import lax
from jax.experimental import pallas as pl
from jax.experimental.pallas import tpu as pltpu

_VARIANT = os.environ.get("KVARIANT", "full")


def kernel(x, router, W1, W2):
    t_loc, d = x.shape
    e_loc, _, f = W1.shape
    t = 2 * t_loc

    def body(x_ref, r_ref, w1_ref, w2_ref, out_ref,
             xpeer_ref, rpeer_ref, cb_ref, partial_ref,
             send_sems, recv_sems):
        my_x = lax.axis_index("x")
        my_y = lax.axis_index("y")
        my_z = lax.axis_index("z")
        peer = (1 - my_x, my_y, my_z)
        mesh_t = pl.DeviceIdType.MESH

        barrier_sem = pltpu.get_barrier_semaphore()
        pl.semaphore_signal(barrier_sem, inc=1, device_id=peer,
                            device_id_type=mesh_t)
        pl.semaphore_wait(barrier_sem, 1)

        if _VARIANT != "no_comm":
            rdma_x = pltpu.make_async_remote_copy(
                src_ref=x_ref, dst_ref=xpeer_ref,
                send_sem=send_sems.at[0], recv_sem=recv_sems.at[0],
                device_id=peer, device_id_type=mesh_t)
            rdma_x.start()
            rdma_r = pltpu.make_async_remote_copy(
                src_ref=r_ref, dst_ref=rpeer_ref,
                send_sem=send_sems.at[1], recv_sem=recv_sems.at[1],
                device_id=peer, device_id_type=mesh_t)
            rdma_r.start()
            rdma_x.wait()
            rdma_r.wait()

        X = jnp.concatenate([x_ref[...], xpeer_ref[...]], axis=0)
        gates = jnp.concatenate(
            [jnp.dot(X, r_ref[...], preferred_element_type=jnp.float32),
             jnp.dot(X, rpeer_ref[...], preferred_element_type=jnp.float32)],
            axis=1)

        eidx = lax.broadcasted_iota(jnp.int32, (t, 4), 1)
        m1 = jnp.max(gates, axis=1, keepdims=True)
        i1 = jnp.min(jnp.where(gates == m1, eidx, 4), axis=1, keepdims=True)
        masked = jnp.where(eidx == i1, -jnp.inf, gates)
        m2 = jnp.max(masked, axis=1, keepdims=True)
        i2 = jnp.min(jnp.where(masked == m2, eidx, 4), axis=1, keepdims=True)
        b = jnp.exp(m2 - m1)
        w_top = 1.0 / (1.0 + b)
        w_sec = b / (1.0 + b)

        Xb = X.astype(jnp.bfloat16)
        acc = jnp.zeros((t, d), jnp.float32)
        if _VARIANT != "no_ffn":
            for j in range(e_loc):
                wj = (jnp.where(i1 == j, w_top, 0.0)
                      + jnp.where(i2 == j, w_sec, 0.0))
                h = jnp.maximum(
                    jnp.dot(Xb, w1_ref[j].astype(jnp.bfloat16),
                            preferred_element_type=jnp.float32),
                    0.0).astype(jnp.bfloat16)
                acc = acc + wj * jnp.dot(
                    h, w2_ref[j].astype(jnp.bfloat16),
                    preferred_element_type=jnp.float32)
        else:
            acc = acc + gates[:, :1]

        if _VARIANT in ("full", "no_ffn"):
            cb_ref[...] = acc[t_loc:, :]
            rdma_c = pltpu.make_async_remote_copy(
                src_ref=cb_ref, dst_ref=partial_ref,
                send_sem=send_sems.at[2], recv_sem=recv_sems.at[2],
                device_id=peer, device_id_type=mesh_t)
            rdma_c.start()
            rdma_c.wait()
            out_ref[...] = acc[:t_loc, :] + partial_ref[...]
        else:
            out_ref[...] = acc[:t_loc, :]

    return pl.pallas_call(
        body,
        out_shape=jax.ShapeDtypeStruct((t_loc, d), jnp.float32),
        in_specs=[pl.BlockSpec(memory_space=pltpu.VMEM)] * 4,
        out_specs=pl.BlockSpec(memory_space=pltpu.VMEM),
        scratch_shapes=[
            pltpu.VMEM((t_loc, d), jnp.float32),
            pltpu.VMEM((d, e_loc), jnp.float32),
            pltpu.VMEM((t_loc, d), jnp.float32),
            pltpu.VMEM((t_loc, d), jnp.float32),
            pltpu.SemaphoreType.DMA((3,)),
            pltpu.SemaphoreType.DMA((3,)),
        ],
        compiler_params=pltpu.CompilerParams(collective_id=0),
    )(x, router, W1, W2)


# device time: 26288 ns/iter; 1.1468x vs baseline; 1.0579x over previous
import os

import jax
import jax.numpy as jnp
from jax import lax
from jax.experimental import pallas as pl
from jax.experimental.pallas import tpu as pltpu

_VARIANT = os.environ.get("KVARIANT", "full")
_F32MM = "f32mm" in _VARIANT


def kernel(x, router, W1, W2):
    t_loc, d = x.shape
    e_loc, _, f = W1.shape

    def body(x_ref, r_ref, w1_ref, w2_ref, out_ref,
             xsend_ref, xpeer_ref, rpeer_ref, wsend_ref, wrecv_ref,
             cb_ref, partial_ref, send_sems, recv_sems):
        my_x = lax.axis_index("x")
        my_y = lax.axis_index("y")
        my_z = lax.axis_index("z")
        peer = (1 - my_x, my_y, my_z)
        mesh_t = pl.DeviceIdType.MESH

        barrier_sem = pltpu.get_barrier_semaphore()
        pl.semaphore_signal(barrier_sem, inc=1, device_id=peer,
                            device_id_type=mesh_t)
        pl.semaphore_wait(barrier_sem, 1)

        xsend_ref[...] = x_ref[...].astype(jnp.bfloat16)
        rdma_x = pltpu.make_async_remote_copy(
            src_ref=xsend_ref, dst_ref=xpeer_ref,
            send_sem=send_sems.at[0], recv_sem=recv_sems.at[0],
            device_id=peer, device_id_type=mesh_t)
        rdma_x.start()
        rdma_r = pltpu.make_async_remote_copy(
            src_ref=r_ref, dst_ref=rpeer_ref,
            send_sem=send_sems.at[1], recv_sem=recv_sems.at[1],
            device_id=peer, device_id_type=mesh_t)
        rdma_r.start()
        rdma_r.wait()

        gates = jnp.concatenate(
            [jnp.dot(x_ref[...], r_ref[...],
                     preferred_element_type=jnp.float32),
             jnp.dot(x_ref[...], rpeer_ref[...],
                     preferred_element_type=jnp.float32)],
            axis=1)
        eidx = lax.broadcasted_iota(jnp.int32, (t_loc, 4), 1)
        m1 = jnp.max(gates, axis=1, keepdims=True)
        i1 = jnp.min(jnp.where(gates == m1, eidx, 4), axis=1, keepdims=True)
        masked = jnp.where(eidx == i1, -jnp.inf, gates)
        m2 = jnp.max(masked, axis=1, keepdims=True)
        i2 = jnp.min(jnp.where(masked == m2, eidx, 4), axis=1, keepdims=True)
        b = jnp.exp(m2 - m1)
        w_top = 1.0 / (1.0 + b)
        w_sec = b / (1.0 + b)

        def wcol(c):
            return (jnp.where(i1 == c, w_top, 0.0)
                    + jnp.where(i2 == c, w_sec, 0.0))

        wsend_ref[...] = jnp.concatenate([wcol(2), wcol(3)], axis=1)
        rdma_w = pltpu.make_async_remote_copy(
            src_ref=wsend_ref, dst_ref=wrecv_ref,
            send_sem=send_sems.at[2], recv_sem=recv_sems.at[2],
            device_id=peer, device_id_type=mesh_t)
        rdma_w.start()

        def ffn(xin, wcols):
            acc = jnp.zeros((t_loc, d), jnp.float32)
            for j in range(e_loc):
                w1j = w1_ref[j] if _F32MM else w1_ref[j].astype(jnp.bfloat16)
                w2j = w2_ref[j] if _F32MM else w2_ref[j].astype(jnp.bfloat16)
                h = jnp.maximum(
                    jnp.dot(xin, w1j, preferred_element_type=jnp.float32),
                    0.0)
                if not _F32MM:
                    h = h.astype(jnp.bfloat16)
                acc = acc + wcols[j] * jnp.dot(
                    h, w2j, preferred_element_type=jnp.float32)
            return acc

        rdma_x.wait()
        rdma_w.wait()
        xp = xpeer_ref[...]
        if _F32MM:
            xp = xp.astype(jnp.float32)
        acc_b = ffn(xp, [wrecv_ref[:, 0:1], wrecv_ref[:, 1:2]])
        cb_ref[...] = acc_b.astype(jnp.bfloat16)
        rdma_c = pltpu.make_async_remote_copy(
            src_ref=cb_ref, dst_ref=partial_ref,
            send_sem=send_sems.at[3], recv_sem=recv_sems.at[3],
            device_id=peer, device_id_type=mesh_t)
        rdma_c.start()

        xa = x_ref[...] if _F32MM else xsend_ref[...]
        acc_a = ffn(xa, [wcol(0), wcol(1)])

        rdma_c.wait()
        out_ref[...] = acc_a + partial_ref[...].astype(jnp.float32)

    return pl.pallas_call(
        body,
        out_shape=jax.ShapeDtypeStruct((t_loc, d), jnp.float32),
        in_specs=[pl.BlockSpec(memory_space=pltpu.VMEM)] * 4,
        out_specs=pl.BlockSpec(memory_space=pltpu.VMEM),
        scratch_shapes=[
            pltpu.VMEM((t_loc, d), jnp.bfloat16),
            pltpu.VMEM((t_loc, d), jnp.bfloat16),
            pltpu.VMEM((d, e_loc), jnp.float32),
            pltpu.VMEM((t_loc, e_loc), jnp.float32),
            pltpu.VMEM((t_loc, e_loc), jnp.float32),
            pltpu.VMEM((t_loc, d), jnp.bfloat16),
            pltpu.VMEM((t_loc, d), jnp.bfloat16),
            pltpu.SemaphoreType.DMA((4,)),
            pltpu.SemaphoreType.DMA((4,)),
        ],
        compiler_params=pltpu.CompilerParams(collective_id=0),
    )(x, router, W1, W2)


# device time: 19950 ns/iter; 1.5112x vs baseline; 1.3177x over previous
import os

import jax
import jax.numpy as jnp
from jax import lax
from jax.experimental import pallas as pl
from jax.experimental.pallas import tpu as pltpu

_VARIANT = os.environ.get("KVARIANT", "full")
_F32MM = "f32mm" in _VARIANT

_N_CHUNKS = 2


def kernel(x, router, W1, W2):
    t_loc, d = x.shape
    e_loc, _, f = W1.shape
    t_ck = t_loc // _N_CHUNKS

    def body(x_ref, rt_ref, w1_ref, w2_ref, out_ref,
             xsend_ref, xpeer_ref, rtpeer_ref, wsend_ref, wrecv_ref,
             cb_ref, partial_ref, send_sems, recv_sems):
        my_x = lax.axis_index("x")
        my_y = lax.axis_index("y")
        my_z = lax.axis_index("z")
        peer = (1 - my_x, my_y, my_z)
        mesh_t = pl.DeviceIdType.MESH

        barrier_sem = pltpu.get_barrier_semaphore()
        pl.semaphore_signal(barrier_sem, inc=1, device_id=peer,
                            device_id_type=mesh_t)
        pl.semaphore_wait(barrier_sem, 1)

        rdma_r = pltpu.make_async_remote_copy(
            src_ref=rt_ref, dst_ref=rtpeer_ref,
            send_sem=send_sems.at[1], recv_sem=recv_sems.at[1],
            device_id=peer, device_id_type=mesh_t)
        rdma_r.start()
        xsend_ref[...] = x_ref[...].astype(jnp.bfloat16)
        rdma_x = pltpu.make_async_remote_copy(
            src_ref=xsend_ref, dst_ref=xpeer_ref,
            send_sem=send_sems.at[0], recv_sem=recv_sems.at[0],
            device_id=peer, device_id_type=mesh_t)
        rdma_x.start()
        rdma_r.wait()

        dn = (((1,), (1,)), ((), ()))
        gates = jnp.concatenate(
            [lax.dot_general(x_ref[...], rt_ref[...], dn,
                             preferred_element_type=jnp.float32),
             lax.dot_general(x_ref[...], rtpeer_ref[...], dn,
                             preferred_element_type=jnp.float32)],
            axis=1)
        eidx = lax.broadcasted_iota(jnp.int32, (t_loc, 4), 1)
        m1 = jnp.max(gates, axis=1, keepdims=True)
        i1 = jnp.min(jnp.where(gates == m1, eidx, 4), axis=1, keepdims=True)
        masked = jnp.where(eidx == i1, -jnp.inf, gates)
        m2 = jnp.max(masked, axis=1, keepdims=True)
        i2 = jnp.min(jnp.where(masked == m2, eidx, 4), axis=1, keepdims=True)
        b = jnp.exp(m2 - m1)
        w_top = 1.0 / (1.0 + b)
        w_sec = b / (1.0 + b)

        def wcol(c):
            return (jnp.where(i1 == c, w_top, 0.0)
                    + jnp.where(i2 == c, w_sec, 0.0))

        wsend_ref[...] = jnp.concatenate([wcol(2), wcol(3)], axis=1)
        rdma_w = pltpu.make_async_remote_copy(
            src_ref=wsend_ref, dst_ref=wrecv_ref,
            send_sem=send_sems.at[2], recv_sem=recv_sems.at[2],
            device_id=peer, device_id_type=mesh_t)
        rdma_w.start()

        def expert(xin, j):
            w1j = w1_ref[j] if _F32MM else w1_ref[j].astype(jnp.bfloat16)
            w2j = w2_ref[j] if _F32MM else w2_ref[j].astype(jnp.bfloat16)
            h = jnp.maximum(
                jnp.dot(xin, w1j, preferred_element_type=jnp.float32), 0.0)
            if not _F32MM:
                h = h.astype(jnp.bfloat16)
            return jnp.dot(h, w2j, preferred_element_type=jnp.float32)

        xa = x_ref[...] if _F32MM else xsend_ref[...]

        acc_a = wcol(0) * expert(xa, 0)

        rdma_x.wait()
        rdma_w.wait()

        rdma_cs = []
        for ck in range(_N_CHUNKS):
            rows = pl.ds(ck * t_ck, t_ck)
            xp = xpeer_ref[rows, :]
            if _F32MM:
                xp = xp.astype(jnp.float32)
            acc_b = (wrecv_ref[rows, 0:1] * expert(xp, 0)
                     + wrecv_ref[rows, 1:2] * expert(xp, 1))
            cb_ref[rows, :] = acc_b.astype(jnp.bfloat16)
            rdma_c = pltpu.make_async_remote_copy(
                src_ref=cb_ref.at[rows, :], dst_ref=partial_ref.at[rows, :],
                send_sem=send_sems.at[3 + ck], recv_sem=recv_sems.at[3 + ck],
                device_id=peer, device_id_type=mesh_t)
            rdma_c.start()
            rdma_cs.append(rdma_c)

        acc_a = acc_a + wcol(1) * expert(xa, 1)

        for rdma_c in rdma_cs:
            rdma_c.wait()
        out_ref[...] = acc_a + partial_ref[...].astype(jnp.float32)

    return pl.pallas_call(
        body,
        out_shape=jax.ShapeDtypeStruct((t_loc, d), jnp.float32),
        in_specs=[pl.BlockSpec(memory_space=pltpu.VMEM)] * 4,
        out_specs=pl.BlockSpec(memory_space=pltpu.VMEM),
        scratch_shapes=[
            pltpu.VMEM((t_loc, d), jnp.bfloat16),
            pltpu.VMEM((t_loc, d), jnp.bfloat16),
            pltpu.VMEM((e_loc, d), jnp.float32),
            pltpu.VMEM((t_loc, e_loc), jnp.float32),
            pltpu.VMEM((t_loc, e_loc), jnp.float32),
            pltpu.VMEM((t_loc, d), jnp.bfloat16),
            pltpu.VMEM((t_loc, d), jnp.bfloat16),
            pltpu.SemaphoreType.DMA((3 + _N_CHUNKS,)),
            pltpu.SemaphoreType.DMA((3 + _N_CHUNKS,)),
        ],
        compiler_params=pltpu.CompilerParams(collective_id=0),
    )(x, router.T, W1, W2)
